# Initial kernel scaffold; baseline (speedup 1.0000x reference)
import jax
import jax.numpy as jnp
from jax.experimental import pallas as pl


def kernel(feat_ref, flow, W1, b1, W2, b2):
    raise NotImplementedError("write your pallas kernel here")


# speedup vs baseline: 1.0405x; 1.0405x over previous
import functools

import jax
import jax.numpy as jnp
from jax import lax
from jax.experimental import pallas as pl
from jax.experimental.pallas import tpu as pltpu
from jax.experimental.pallas import tpu_sc as plsc

B, C, H, W = 4, 128, 128, 128
HW = H * W
HID = 64


def _shift3(x, sy, sx):
    c, h, w = x.shape
    if sx == 1:
        x = jnp.concatenate([x[:, :, 1:], jnp.zeros((c, h, 1), x.dtype)], axis=2)
    elif sx == -1:
        x = jnp.concatenate([jnp.zeros((c, h, 1), x.dtype), x[:, :, :-1]], axis=2)
    if sy == 1:
        x = jnp.concatenate([x[:, 1:, :], jnp.zeros((c, 1, w), x.dtype)], axis=1)
    elif sy == -1:
        x = jnp.concatenate([jnp.zeros((c, 1, w), x.dtype), x[:, :-1, :]], axis=1)
    return x


def _prep_body(feat_ref, flow_ref, w1_ref, b1_ref, w2_ref, b2_ref, idx_ref, wts_ref):
    feat = feat_ref[0]
    acc = jnp.zeros((HID, HW), jnp.float32)
    for dy in range(3):
        for dx in range(3):
            xt = _shift3(feat, dy - 1, dx - 1).reshape(C, HW)
            w1s = w1_ref[pl.ds((dy * 3 + dx) * C, C), :]
            acc = acc + lax.dot_general(
                w1s, xt, (((0,), (0,)), ((), ())),
                preferred_element_type=jnp.float32)
    h = jnp.maximum(acc + b1_ref[...], 0.0)
    h3 = h.reshape(HID, H, W)
    m = jnp.zeros((1, HW), jnp.float32)
    for dy in range(3):
        for dx in range(3):
            ht = _shift3(h3, dy - 1, dx - 1).reshape(HID, HW)
            w2s = w2_ref[pl.ds(dy * 3 + dx, 1), :]
            m = m + lax.dot_general(
                w2s, ht, (((1,), (0,)), ((), ())),
                preferred_element_type=jnp.float32)
    e2 = jnp.exp(m + b2_ref[...]).reshape(H, W)

    flow0 = flow_ref[0, 0]
    flow1 = flow_ref[0, 1]
    gx = lax.broadcasted_iota(jnp.int32, (H, W), 1).astype(jnp.float32)
    gy = lax.broadcasted_iota(jnp.int32, (H, W), 0).astype(jnp.float32)
    fx = gx + flow0
    fy = gy + flow1
    x0 = jnp.floor(fx).astype(jnp.int32)
    y0 = jnp.floor(fy).astype(jnp.int32)
    x1 = x0 + 1
    y1 = y0 + 1
    x0f = x0.astype(jnp.float32)
    x1f = x1.astype(jnp.float32)
    y0f = y0.astype(jnp.float32)
    y1f = y1.astype(jnp.float32)
    wNW = (x1f - fx) * (y1f - fy)
    wNE = (fx - x0f) * (y1f - fy)
    wSW = (x1f - fx) * (fy - y0f)
    wSE = (fx - x0f) * (fy - y0f)
    lins = []
    wts = []
    for xi, yi, wgt in ((x0, y0, wNW), (x1, y0, wNE), (x0, y1, wSW), (x1, y1, wSE)):
        valid = (xi >= 0) & (xi < W) & (yi >= 0) & (yi < H)
        w_ = jnp.where(valid, wgt, 0.0) * e2
        lin = jnp.clip(yi, 0, H - 1) * W + jnp.clip(xi, 0, W - 1)
        lins.append(lin.reshape(1, HW))
        wts.append(w_.reshape(1, HW))
    idx_ref[0] = jnp.concatenate(lins, axis=0)
    wts_ref[0] = jnp.concatenate(wts, axis=0)


def _prep(feat, flow, w1r, b1r, w2r, b2r):
    return pl.pallas_call(
        _prep_body,
        grid=(B,),
        in_specs=[
            pl.BlockSpec((1, C, H, W), lambda b: (b, 0, 0, 0)),
            pl.BlockSpec((1, 2, H, W), lambda b: (b, 0, 0, 0)),
            pl.BlockSpec((9 * C, HID), lambda b: (0, 0)),
            pl.BlockSpec((HID, 1), lambda b: (0, 0)),
            pl.BlockSpec((9, HID), lambda b: (0, 0)),
            pl.BlockSpec((1, 1), lambda b: (0, 0)),
        ],
        out_specs=[
            pl.BlockSpec((1, 4, HW), lambda b: (b, 0, 0)),
            pl.BlockSpec((1, 4, HW), lambda b: (b, 0, 0)),
        ],
        out_shape=[
            jax.ShapeDtypeStruct((B, 4, HW), jnp.int32),
            jax.ShapeDtypeStruct((B, 4, HW), jnp.float32),
        ],
    )(feat, flow, w1r, b1r, w2r, b2r)


def kernel(feat_ref, flow, W1, b1, W2, b2):
    w1r = W1.transpose(2, 3, 1, 0).reshape(9 * C, HID)
    b1r = b1.reshape(HID, 1)
    w2r = W2.transpose(2, 3, 1, 0).reshape(9, HID)
    b2r = b2.reshape(1, 1)
    idx, wts = _prep(feat_ref, flow, w1r, b1r, w2r, b2r)

    featT = feat_ref.reshape(B, C, HW)
    out = jnp.zeros((B, HW, C + 1), jnp.float32)
    src = jnp.concatenate([featT, jnp.ones((B, 1, HW), jnp.float32)], axis=1)
    for k in range(4):
        vals = src * wts[:, k][:, None, :]
        out = out.at[jnp.arange(B)[:, None], idx[:, k]].add(vals.transpose(0, 2, 1))
    norm = out[:, :, C:]
    norm = jnp.where(norm == 0.0, 1.0, norm)
    res = out[:, :, :C] / norm
    return res.reshape(B, H, W, C).transpose(0, 3, 1, 2)


# speedup vs baseline: 2.3865x; 2.2936x over previous
import functools

import jax
import jax.numpy as jnp
from jax import lax
from jax.experimental import pallas as pl
from jax.experimental.pallas import tpu as pltpu
from jax.experimental.pallas import tpu_sc as plsc

B, C, H, W = 4, 128, 128, 128
HW = H * W
HID = 64


def _shift3(x, sy, sx):
    c, h, w = x.shape
    if sx == 1:
        x = jnp.concatenate([x[:, :, 1:], jnp.zeros((c, h, 1), x.dtype)], axis=2)
    elif sx == -1:
        x = jnp.concatenate([jnp.zeros((c, h, 1), x.dtype), x[:, :, :-1]], axis=2)
    if sy == 1:
        x = jnp.concatenate([x[:, 1:, :], jnp.zeros((c, 1, w), x.dtype)], axis=1)
    elif sy == -1:
        x = jnp.concatenate([jnp.zeros((c, 1, w), x.dtype), x[:, :-1, :]], axis=1)
    return x


def _conv_body(feat_ref, w1_ref, b1_ref, w2_ref, b2_ref, e_ref):
    feat2d = feat_ref[0].reshape(C, HW)
    acc = jnp.zeros((HID, HW), jnp.float32)
    for dy in range(3):
        for dx in range(3):
            w1s = w1_ref[pl.ds((dy * 3 + dx) * C, C), :]
            m1 = lax.dot_general(
                w1s, feat2d, (((0,), (0,)), ((), ())),
                preferred_element_type=jnp.float32)
            acc = acc + _shift3(m1.reshape(HID, H, W),
                                dy - 1, dx - 1).reshape(HID, HW)
    h = jnp.maximum(acc + b1_ref[...], 0.0)
    m = jnp.zeros((1, HW), jnp.float32)
    for dy in range(3):
        for dx in range(3):
            w2s = w2_ref[pl.ds(dy * 3 + dx, 1), :]
            m2 = lax.dot_general(
                w2s, h, (((1,), (0,)), ((), ())),
                preferred_element_type=jnp.float32)
            m = m + _shift3(m2.reshape(1, H, W), dy - 1, dx - 1).reshape(1, HW)
    e_ref[...] = jnp.exp(m + b2_ref[...]).reshape(1, 1, HW)


def _conv(feat, w1r, b1r, w2r, b2r):
    return pl.pallas_call(
        _conv_body,
        grid=(B,),
        compiler_params=pltpu.CompilerParams(
            vmem_limit_bytes=100 * 1024 * 1024),
        in_specs=[
            pl.BlockSpec((1, C, H, W), lambda b: (b, 0, 0, 0)),
            pl.BlockSpec((9 * C, HID), lambda b: (0, 0)),
            pl.BlockSpec((HID, 1), lambda b: (0, 0)),
            pl.BlockSpec((9, HID), lambda b: (0, 0)),
            pl.BlockSpec((1, 1), lambda b: (0, 0)),
        ],
        out_specs=pl.BlockSpec((1, 1, HW), lambda b: (b, 0, 0)),
        out_shape=jax.ShapeDtypeStruct((B, 1, HW), jnp.float32),
    )(feat, w1r, b1r, w2r, b2r)


def _wts_body(flow_ref, e_ref, idx_ref, wts_ref, norm_ref):
    e2 = e_ref[0, 0].reshape(H, W)
    flow0 = flow_ref[0, 0]
    flow1 = flow_ref[0, 1]
    gx = lax.broadcasted_iota(jnp.int32, (H, W), 1).astype(jnp.float32)
    gy = lax.broadcasted_iota(jnp.int32, (H, W), 0).astype(jnp.float32)
    fx = gx + flow0
    fy = gy + flow1
    x0 = jnp.floor(fx).astype(jnp.int32)
    y0 = jnp.floor(fy).astype(jnp.int32)
    x1 = x0 + 1
    y1 = y0 + 1
    x0f = x0.astype(jnp.float32)
    x1f = x1.astype(jnp.float32)
    y0f = y0.astype(jnp.float32)
    y1f = y1.astype(jnp.float32)
    wNW = (x1f - fx) * (y1f - fy)
    wNE = (fx - x0f) * (y1f - fy)
    wSW = (x1f - fx) * (fy - y0f)
    wSE = (fx - x0f) * (fy - y0f)
    lins = []
    wts = []
    norm2d = jnp.zeros((H, W), jnp.float32)
    for xi, yi, wgt in ((x0, y0, wNW), (x1, y0, wNE), (x0, y1, wSW), (x1, y1, wSE)):
        valid = (xi >= 0) & (xi < W) & (yi >= 0) & (yi < H)
        w_ = jnp.where(valid, wgt, 0.0) * e2
        ycl = jnp.clip(yi, 0, H - 1)
        xcl = jnp.clip(xi, 0, W - 1)
        lin = ycl * W + xcl
        lins.append(lin.reshape(1, HW))
        wts.append(w_.reshape(1, HW))
        for ci in range(H // 16):
            r0 = ci * 16
            wc = w_[r0:r0 + 16].astype(jnp.bfloat16)[:, :, None]
            dyc = (lax.broadcasted_iota(jnp.int32, (16, W, H), 2)
                   - ycl[r0:r0 + 16][:, :, None])
            oy = jnp.clip(1 - jnp.abs(dyc), 0, 1).astype(jnp.bfloat16)
            ayc = (oy * wc).reshape(16 * W, H)
            dxc = (lax.broadcasted_iota(jnp.int32, (16, W, W), 2)
                   - xcl[r0:r0 + 16][:, :, None])
            bxc = jnp.clip(1 - jnp.abs(dxc), 0, 1).astype(jnp.bfloat16).reshape(16 * W, W)
            norm2d = norm2d + lax.dot_general(
                ayc, bxc, (((0,), (0,)), ((), ())),
                preferred_element_type=jnp.float32)
    idx_ref[0] = jnp.concatenate(lins, axis=0)
    wts_ref[0] = jnp.concatenate(wts, axis=0)
    norm_ref[...] = norm2d.reshape(1, 1, HW)


def _wts(flow, e):
    return pl.pallas_call(
        _wts_body,
        grid=(B,),
        in_specs=[
            pl.BlockSpec((1, 2, H, W), lambda b: (b, 0, 0, 0)),
            pl.BlockSpec((1, 1, HW), lambda b: (b, 0, 0)),
        ],
        out_specs=[
            pl.BlockSpec((1, 4, HW), lambda b: (b, 0, 0)),
            pl.BlockSpec((1, 4, HW), lambda b: (b, 0, 0)),
            pl.BlockSpec((1, 1, HW), lambda b: (b, 0, 0)),
        ],
        out_shape=[
            jax.ShapeDtypeStruct((B, 4, HW), jnp.int32),
            jax.ShapeDtypeStruct((B, 4, HW), jnp.float32),
            jax.ShapeDtypeStruct((B, 1, HW), jnp.float32),
        ],
    )(flow, e)


CH_HALF = 64
BAND = 1024
GRP = 64
NG = BAND // GRP
SLAB = 512


def _splat_body(feat_hbm, wts_hbm, lin_hbm, zeros_hbm, out_hbm,
                acc, featv, wv, lv, iv, stage):
    c = lax.axis_index("c")
    s = lax.axis_index("s")
    band0 = s * BAND
    dest0 = s * SLAB
    zeros16 = jnp.zeros((16,), jnp.float32)

    def batch_body(b, carry):
        pltpu.sync_copy(zeros_hbm, acc.at[pl.ds(dest0, SLAB), :])
        plsc.subcore_barrier()
        pltpu.sync_copy(wts_hbm.at[b, :, pl.ds(band0, BAND)], wv)
        pltpu.sync_copy(lin_hbm.at[b, :, pl.ds(band0, BAND)], lv)

        def trg(g, tcarry):
            for k in range(4):
                for h4 in range(4):
                    vvv = lv[k, pl.ds(g * GRP + h4 * 16, 16)]
                    iv[k, g, pl.ds(h4 * 16, 16)] = lax.shift_right_logical(vvv, 1)
            return tcarry

        lax.fori_loop(0, NG, trg, 0)

        def group_body(g, gcarry):
            pltpu.sync_copy(
                feat_hbm.at[c, b, pl.ds(band0 + g * GRP, GRP), :], featv)
            for k in range(4):
                def h_body(hh, hcarry):
                    i0 = hh * 16
                    wk16 = wv[k, pl.ds(g * GRP + i0, 16)]
                    lv16 = lv[k, pl.ds(g * GRP + i0, 16)]
                    off16 = (lv16 & 1) * 64
                    for j in range(16):
                        p = i0 + j
                        bc = lax.broadcast(wk16[j], (16,))
                        off = off16[j]
                        noff = 64 - off
                        fvs = [featv[p, pl.ds(cb * 16, 16)] for cb in range(4)]
                        for cb in range(4):
                            stage[p, pl.ds(off + cb * 16, 16)] = fvs[cb] * bc
                        for cb in range(4):
                            stage[p, pl.ds(noff + cb * 16, 16)] = zeros16
                    return hcarry

                lax.fori_loop(0, GRP // 16, h_body, 0)
                pltpu.sync_copy(stage, acc.at[iv.at[k, g]], add=True)
            return gcarry

        lax.fori_loop(0, NG, group_body, 0)
        plsc.subcore_barrier()
        pltpu.sync_copy(acc.at[pl.ds(dest0, SLAB), :],
                        out_hbm.at[b, c, pl.ds(dest0, SLAB), :])
        return carry

    lax.fori_loop(0, B, batch_body, 0)


@functools.lru_cache(maxsize=1)
def _get_splat():
    return pl.kernel(
        _splat_body,
        out_type=jax.ShapeDtypeStruct((B, 2, HW // 2, 128), jnp.float32),
        mesh=plsc.VectorSubcoreMesh(core_axis_name="c", subcore_axis_name="s"),
        scratch_types=[
            pltpu.VMEM_SHARED((HW // 2, 128), jnp.float32),
            pltpu.VMEM((GRP, CH_HALF), jnp.float32),
            pltpu.VMEM((4, BAND), jnp.float32),
            pltpu.VMEM((4, BAND), jnp.int32),
            pltpu.VMEM((4, NG, GRP), jnp.int32),
            pltpu.VMEM((GRP, 128), jnp.float32),
        ],
    )


def _div_body(raw_ref, norm_ref, out_ref):
    nv = norm_ref[0, 0]
    den = jnp.where(nv == 0.0, 1.0, nv)
    inv = (1.0 / den)[:, None]
    for cc in range(2):
        out_ref[0, cc] = raw_ref[0, cc] * inv


def _div(raw, norm):
    return pl.pallas_call(
        _div_body,
        grid=(B, 8),
        in_specs=[
            pl.BlockSpec((1, 2, HW // 8, CH_HALF), lambda b, q: (b, 0, q, 0)),
            pl.BlockSpec((1, 1, HW // 8), lambda b, q: (b, 0, q)),
        ],
        out_specs=pl.BlockSpec((1, 2, HW // 8, CH_HALF), lambda b, q: (b, 0, q, 0)),
        out_shape=jax.ShapeDtypeStruct((B, 2, HW, CH_HALF), jnp.float32),
    )(raw, norm)


def kernel(feat_ref, flow, W1, b1, W2, b2):
    w1r = W1.transpose(2, 3, 1, 0).reshape(9 * C, HID)
    b1r = b1.reshape(HID, 1)
    w2r = W2.transpose(2, 3, 1, 0).reshape(9, HID)
    b2r = b2.reshape(1, 1)
    e = _conv(feat_ref, w1r, b1r, w2r, b2r)
    lin, wts, norm = _wts(flow, e)
    featT = feat_ref.reshape(B, 2, CH_HALF, HW).transpose(1, 0, 3, 2)
    zeros = jnp.zeros((SLAB, 128), jnp.float32)
    raw = _get_splat()(featT, wts, lin, zeros)
    out = _div(raw.reshape(B, 2, HW, CH_HALF), norm)
    return (out.reshape(B, 2, H, W, CH_HALF)
            .transpose(0, 1, 4, 2, 3).reshape(B, C, H, W))
